# baseline (speedup 1.0000x reference)
import functools

import jax
import jax.numpy as jnp
from jax import lax
from jax.experimental import pallas as pl
from jax.experimental.pallas import tpu as pltpu
from jax.experimental.pallas import tpu_sc as plsc

B, N, Fw, D, K, H = 2, 10000, 64, 64, 32, 1
BN = B * N
EPS = 1e-5
ROWS_T = 128
NP = 10112
NEG = -3.0e38


NB = NP // 128
CD = 8


def _graph_body(wt_ref, wall_ref, nt_ref, nall_ref, idx_ref, s_ref):
    s = jax.lax.dot_general(wt_ref[...], wall_ref[...],
                            (((1,), (1,)), ((), ())),
                            preferred_element_type=jnp.float32)
    s = s / (nt_ref[...] * nall_ref[...])
    lane = jax.lax.broadcasted_iota(jnp.int32, (ROWS_T, NP), 1)
    s_ref[...] = jnp.where(lane < N, s, NEG)
    lane128 = jax.lax.broadcasted_iota(jnp.int32, (ROWS_T, 128), 1)

    cv = [jnp.full((ROWS_T, 128), NEG, jnp.float32) for _ in range(CD)]
    cl = [jnp.zeros((ROWS_T, 128), jnp.int32) for _ in range(CD)]
    for c in range(NB):
        blk = s_ref[:, c * 128:(c + 1) * 128]
        onehot_c = lane128 == c
        for l in range(CD):
            m = jnp.max(blk, axis=1, keepdims=True)
            li = jnp.min(jnp.where(blk == m, lane128, 128), axis=1,
                         keepdims=True)
            cv[l] = jnp.where(onehot_c, m, cv[l])
            cl[l] = jnp.where(onehot_c, li, cl[l])
            blk = jnp.where(lane128 == li, NEG, blk)

    def sel(k, st):
        acc, cvt, clt = st
        bm = cvt[0]
        m = jnp.max(bm, axis=1, keepdims=True)
        cstar = jnp.min(jnp.where(bm == m, lane128, 128), axis=1,
                        keepdims=True)
        oh = lane128 == cstar
        lanein = jnp.sum(jnp.where(oh, clt[0], 0), axis=1, keepdims=True)
        acc = jnp.where(lane128 == k, cstar * 128 + lanein, acc)
        cvt = tuple(jnp.where(oh, cvt[l + 1], cvt[l]) for l in range(CD - 1)
                    ) + (jnp.where(oh, NEG, cvt[CD - 1]),)
        clt = tuple(jnp.where(oh, clt[l + 1], clt[l]) for l in range(CD - 1)
                    ) + (clt[CD - 1],)
        return (acc, cvt, clt)

    acc, _, _ = jax.lax.fori_loop(
        0, K, sel,
        (jnp.zeros((ROWS_T, 128), jnp.int32), tuple(cv), tuple(cl)))
    idx_ref[...] = acc[:, :K]


def _cos_topk(emb_W):
    nrm = jnp.linalg.norm(emb_W, axis=-1)
    wall = jnp.pad(emb_W, ((0, NP - N), (0, 0)))
    nall = jnp.pad(nrm, (0, NP - N), constant_values=1.0).reshape(1, NP)
    ncol = nrm.reshape(N, 1)
    grid = N // ROWS_T + (1 if N % ROWS_T else 0)
    wpad = jnp.pad(emb_W, ((0, grid * ROWS_T - N), (0, 0)))
    npad = jnp.pad(ncol, ((0, grid * ROWS_T - N), (0, 0)), constant_values=1.0)
    idx = pl.pallas_call(
        _graph_body,
        grid=(grid,),
        in_specs=[
            pl.BlockSpec((ROWS_T, D), lambda i: (i, 0)),
            pl.BlockSpec((NP, D), lambda i: (0, 0)),
            pl.BlockSpec((ROWS_T, 1), lambda i: (i, 0)),
            pl.BlockSpec((1, NP), lambda i: (0, 0)),
        ],
        out_specs=pl.BlockSpec((ROWS_T, K), lambda i: (i, 0)),
        out_shape=jax.ShapeDtypeStruct((grid * ROWS_T, K), jnp.int32),
        scratch_shapes=[pltpu.VMEM((ROWS_T, NP), jnp.float32)],
        compiler_params=pltpu.CompilerParams(
            dimension_semantics=("arbitrary",)),
    )(wpad, wall, npad, nall)
    return idx[:N]


def _post_body(agg_ref, emb2_ref, bias_ref, g1_ref, b1_ref, g2_ref, b2_ref,
               ow_ref, ob_ref, out_ref):
    out = agg_ref[...] + bias_ref[...]
    mu = jnp.mean(out, axis=0, keepdims=True)
    var = jnp.mean((out - mu) ** 2, axis=0, keepdims=True)
    out = (out - mu) * jax.lax.rsqrt(var + EPS) * g1_ref[...] + b1_ref[...]
    out = jnp.maximum(out, 0.0)
    x3 = out * emb2_ref[...]
    mu2 = jnp.mean(x3, axis=0, keepdims=True)
    var2 = jnp.mean((x3 - mu2) ** 2, axis=0, keepdims=True)
    t = (x3 - mu2) * jax.lax.rsqrt(var2 + EPS) * g2_ref[...] + b2_ref[...]
    t = jnp.maximum(t, 0.0)
    o = jnp.sum(t * ow_ref[...], axis=1) + ob_ref[0, 0]
    out_ref[...] = o


def _postprocess(agg, emb2, bias_gnn, g1, b1, g2, b2, out_W, out_b):
    return pl.pallas_call(
        _post_body,
        out_shape=jax.ShapeDtypeStruct((BN,), jnp.float32),
    )(agg, emb2, bias_gnn.reshape(1, D), g1.reshape(1, D), b1.reshape(1, D),
      g2.reshape(1, D), b2.reshape(1, D), out_W.reshape(1, D),
      out_b.reshape(1, 1))


NW = 32
NODES_W = 640
BNP = NW * NODES_W
CH = 8
CHN = NODES_W // CH
ROWW = 40
RPC = CH * ROWW
XCOL = 128


def _dyn_gather(v, idx):
    dnums = lax.GatherDimensionNumbers(
        offset_dims=(), collapsed_slice_dims=(0,), start_index_map=(0,))
    return lax.gather(v, idx[:, None], dnums, slice_sizes=(1,),
                      mode=lax.GatherScatterMode.PROMISE_IN_BOUNDS)


def _agg_sc_body(xa_hbm, aj_hbm, idx_hbm, out_hbm,
                 idx_v, gbuf, ajbuf, obuf, gs0, gs1):
    wid = lax.axis_index("s") * 2 + lax.axis_index("c")
    base = wid * NODES_W
    pltpu.sync_copy(idx_hbm.at[pl.ds(base * ROWW, NODES_W * ROWW)], idx_v)
    gsems = [gs0, gs1]
    iota = lax.iota(jnp.int32, 16)

    def gather_copies(c, b):
        isl = idx_v.at[pl.ds(c * RPC, RPC)]
        return (pltpu.make_async_copy(xa_hbm.at[isl],
                                      gbuf.at[pl.ds(b * RPC, RPC)], gsems[b]),
                pltpu.make_async_copy(aj_hbm.at[isl],
                                      ajbuf.at[pl.ds(b * RPC, RPC)], gsems[b]))

    for b in range(2):
        for cp in gather_copies(b, b):
            cp.start()

    def compute_node(c, b, n):
        base_r = b * RPC + n * ROWW
        ibase = c * RPC + n * ROWW

        vself = _dyn_gather(idx_v[pl.ds(ibase, 16)], iota * 0)
        nbr1 = idx_v[pl.ds(ibase + 1, 16)]
        nbr2 = idx_v[pl.ds(ibase + 17, 16)]
        aj_s = _dyn_gather(ajbuf[pl.ds(base_r, 16)], iota * 0)
        aj1 = ajbuf[pl.ds(base_r + 1, 16)]
        aj2 = ajbuf[pl.ds(base_r + 17, 16)]
        ai_s = _dyn_gather(gbuf[base_r, pl.ds(64, 16)], iota * 0 + 1)

        def leaky(a):
            return jnp.where(a >= 0, a, 0.2 * a)

        a1 = jnp.where(nbr1 == vself, NEG, leaky(ai_s + aj1))
        a2 = jnp.where(nbr2 == vself, NEG, leaky(ai_s + aj2))
        a_s = leaky(ai_s + aj_s)

        def allred(v, op):
            for s in (8, 4, 2, 1):
                v = op(v, _dyn_gather(v, jnp.bitwise_xor(iota, s)))
            return v

        m = allred(jnp.maximum(jnp.maximum(a1, a2), a_s), jnp.maximum)
        e1 = jnp.exp(a1 - m)
        e2 = jnp.exp(a2 - m)
        e_s = jnp.exp(a_s - m)
        den = allred(e1 + e2, jnp.add) + e_s + 1e-16
        p1 = e1 / den
        p2 = e2 / den
        p_s = e_s / den
        for g in range(4):
            acc = p_s * gbuf[base_r, pl.ds(g * 16, 16)]
            for k in range(16):
                pk = _dyn_gather(p1, iota * 0 + k)
                acc = acc + pk * gbuf[base_r + 1 + k, pl.ds(g * 16, 16)]
            for k in range(16):
                pk = _dyn_gather(p2, iota * 0 + k)
                acc = acc + pk * gbuf[base_r + 17 + k, pl.ds(g * 16, 16)]
            obuf[n, pl.ds(g * 16, 16)] = acc

    def outer(t, carry):
        for b in range(2):
            c = 2 * t + b
            for cp in gather_copies(c, b):
                cp.wait()
            for n in range(CH):
                compute_node(c, b, n)
            pltpu.sync_copy(obuf, out_hbm.at[pl.ds(base + c * CH, CH)])

            @pl.when(c + 2 < CHN)
            def _():
                for cp in gather_copies(c + 2, b):
                    cp.start()
        return carry

    lax.fori_loop(0, CHN // 2, outer, 0)


def _agg_sparsecore(xa, aj_arr, idx_flat):
    mesh = plsc.VectorSubcoreMesh(core_axis_name="c", subcore_axis_name="s")
    k = functools.partial(
        pl.kernel, mesh=mesh,
        out_type=jax.ShapeDtypeStruct((BNP, 64), jnp.float32),
        scratch_types=[
            pltpu.VMEM((NODES_W * ROWW,), jnp.int32),
            pltpu.VMEM((2 * RPC, XCOL), jnp.float32),
            pltpu.VMEM((2 * RPC,), jnp.float32),
            pltpu.VMEM((CH, 64), jnp.float32),
            pltpu.SemaphoreType.DMA,
            pltpu.SemaphoreType.DMA,
        ],
    )(_agg_sc_body)
    return k(xa, aj_arr, idx_flat)


def kernel(data, emb_W, lin_W, att_i, att_j, att_em_i, att_em_j, bias_gnn,
           bn1_gamma, bn1_beta, bn_out_gamma, bn_out_beta, out_W, out_b):
    topk_idx = _cos_topk(emb_W)

    x = data.reshape(BN, Fw) @ lin_W
    av_i = att_i.reshape(D)
    av_j = att_j.reshape(D)
    ae_i = att_em_i.reshape(D)
    ae_j = att_em_j.reshape(D)
    emb_ai = emb_W @ ae_i
    emb_aj = emb_W @ ae_j
    ai = x @ av_i + jnp.tile(emb_ai, B)
    aj = x @ av_j + jnp.tile(emb_aj, B)

    xa = jnp.concatenate(
        [x, aj[:, None], ai[:, None], jnp.zeros((BN, XCOL - D - 2), jnp.float32)],
        axis=1)
    self_ids = jnp.arange(BN, dtype=jnp.int32)
    nbr = jnp.concatenate([topk_idx + b * N for b in range(B)], axis=0)
    idxm = jnp.concatenate(
        [self_ids[:, None], nbr.astype(jnp.int32),
         jnp.tile(self_ids[:, None], (1, ROWW - 1 - K))], axis=1)
    pad_rows = jnp.tile((jnp.arange(BNP - BN, dtype=jnp.int32) * 37 % BN)[:, None],
                        (1, ROWW))
    idx_flat = jnp.concatenate([idxm, pad_rows], axis=0).reshape(-1)
    agg = _agg_sparsecore(xa, aj, idx_flat)[:BN]

    emb2 = jnp.tile(emb_W, (B, 1))
    o = _postprocess(agg, emb2, bias_gnn, bn1_gamma, bn1_beta,
                     bn_out_gamma, bn_out_beta, out_W, out_b)
    return o.reshape(B, N)


# speedup vs baseline: 2.8364x; 2.8364x over previous
import functools

import jax
import jax.numpy as jnp
from jax import lax
from jax.experimental import pallas as pl
from jax.experimental.pallas import tpu as pltpu
from jax.experimental.pallas import tpu_sc as plsc

B, N, Fw, D, K, H = 2, 10000, 64, 64, 32, 1
BN = B * N
EPS = 1e-5
ROWS_T = 256
NP = 10112
NEG = -3.0e38


def _graph_body(wt_ref, wall_ref, nt_ref, nall_ref, idx_ref, s_ref):
    s = jax.lax.dot_general(wt_ref[...], wall_ref[...],
                            (((1,), (1,)), ((), ())),
                            preferred_element_type=jnp.float32)
    s = s / (nt_ref[...] * nall_ref[...])
    lane = jax.lax.broadcasted_iota(jnp.int32, (ROWS_T, NP), 1)
    s_ref[...] = jnp.where(lane < N, s, NEG)

    def body(k, acc):
        sv = s_ref[...]
        m = jnp.max(sv, axis=1, keepdims=True)
        idx = jnp.min(jnp.where(sv == m, lane, NP), axis=1, keepdims=True)
        acc = jnp.where(lane[:, :128] == k, idx, acc)
        s_ref[...] = jnp.where(lane == idx, NEG, sv)
        return acc

    acc = jax.lax.fori_loop(0, K, body,
                            jnp.zeros((ROWS_T, 128), jnp.int32))
    idx_ref[...] = acc[:, :K]


def _cos_topk(emb_W):
    nrm = jnp.linalg.norm(emb_W, axis=-1)
    wall = jnp.pad(emb_W, ((0, NP - N), (0, 0)))
    nall = jnp.pad(nrm, (0, NP - N), constant_values=1.0).reshape(1, NP)
    ncol = nrm.reshape(N, 1)
    grid = N // ROWS_T + (1 if N % ROWS_T else 0)
    wpad = jnp.pad(emb_W, ((0, grid * ROWS_T - N), (0, 0)))
    npad = jnp.pad(ncol, ((0, grid * ROWS_T - N), (0, 0)), constant_values=1.0)
    idx = pl.pallas_call(
        _graph_body,
        grid=(grid,),
        in_specs=[
            pl.BlockSpec((ROWS_T, D), lambda i: (i, 0)),
            pl.BlockSpec((NP, D), lambda i: (0, 0)),
            pl.BlockSpec((ROWS_T, 1), lambda i: (i, 0)),
            pl.BlockSpec((1, NP), lambda i: (0, 0)),
        ],
        out_specs=pl.BlockSpec((ROWS_T, K), lambda i: (i, 0)),
        out_shape=jax.ShapeDtypeStruct((grid * ROWS_T, K), jnp.int32),
        scratch_shapes=[pltpu.VMEM((ROWS_T, NP), jnp.float32)],
        compiler_params=pltpu.CompilerParams(
            dimension_semantics=("arbitrary",)),
    )(wpad, wall, npad, nall)
    return idx[:N]


def _post_body(agg_ref, emb2_ref, bias_ref, g1_ref, b1_ref, g2_ref, b2_ref,
               ow_ref, ob_ref, out_ref):
    out = agg_ref[...] + bias_ref[...]
    mu = jnp.mean(out, axis=0, keepdims=True)
    var = jnp.mean((out - mu) ** 2, axis=0, keepdims=True)
    out = (out - mu) * jax.lax.rsqrt(var + EPS) * g1_ref[...] + b1_ref[...]
    out = jnp.maximum(out, 0.0)
    x3 = out * emb2_ref[...]
    mu2 = jnp.mean(x3, axis=0, keepdims=True)
    var2 = jnp.mean((x3 - mu2) ** 2, axis=0, keepdims=True)
    t = (x3 - mu2) * jax.lax.rsqrt(var2 + EPS) * g2_ref[...] + b2_ref[...]
    t = jnp.maximum(t, 0.0)
    o = jnp.sum(t * ow_ref[...], axis=1) + ob_ref[0, 0]
    out_ref[...] = o


def _postprocess(agg, emb2, bias_gnn, g1, b1, g2, b2, out_W, out_b):
    return pl.pallas_call(
        _post_body,
        out_shape=jax.ShapeDtypeStruct((BN,), jnp.float32),
    )(agg, emb2, bias_gnn.reshape(1, D), g1.reshape(1, D), b1.reshape(1, D),
      g2.reshape(1, D), b2.reshape(1, D), out_W.reshape(1, D),
      out_b.reshape(1, 1))


NW = 32
NODES_W = 640
BNP = NW * NODES_W
CH = 8
CHN = NODES_W // CH
ROWW = 40
RPC = CH * ROWW
XCOL = 128


def _dyn_gather(v, idx):
    dnums = lax.GatherDimensionNumbers(
        offset_dims=(), collapsed_slice_dims=(0,), start_index_map=(0,))
    return lax.gather(v, idx[:, None], dnums, slice_sizes=(1,),
                      mode=lax.GatherScatterMode.PROMISE_IN_BOUNDS)


def _agg_sc_body(xa_hbm, aj_hbm, idx_hbm, out_hbm,
                 idx_v, gbuf, ajbuf, obuf, gs0, gs1):
    wid = lax.axis_index("s") * 2 + lax.axis_index("c")
    base = wid * NODES_W
    pltpu.sync_copy(idx_hbm.at[pl.ds(base * ROWW, NODES_W * ROWW)], idx_v)
    gsems = [gs0, gs1]
    iota = lax.iota(jnp.int32, 16)

    def gather_copies(c, b):
        isl = idx_v.at[pl.ds(c * RPC, RPC)]
        return (pltpu.make_async_copy(xa_hbm.at[isl],
                                      gbuf.at[pl.ds(b * RPC, RPC)], gsems[b]),
                pltpu.make_async_copy(aj_hbm.at[isl],
                                      ajbuf.at[pl.ds(b * RPC, RPC)], gsems[b]))

    for b in range(2):
        for cp in gather_copies(b, b):
            cp.start()

    def compute_node(c, b, n):
        base_r = b * RPC + n * ROWW
        ibase = c * RPC + n * ROWW

        vself = _dyn_gather(idx_v[pl.ds(ibase, 16)], iota * 0)
        nbr1 = idx_v[pl.ds(ibase + 1, 16)]
        nbr2 = idx_v[pl.ds(ibase + 17, 16)]
        aj_s = _dyn_gather(ajbuf[pl.ds(base_r, 16)], iota * 0)
        aj1 = ajbuf[pl.ds(base_r + 1, 16)]
        aj2 = ajbuf[pl.ds(base_r + 17, 16)]
        ai_s = _dyn_gather(gbuf[base_r, pl.ds(64, 16)], iota * 0 + 1)

        def leaky(a):
            return jnp.where(a >= 0, a, 0.2 * a)

        a1 = jnp.where(nbr1 == vself, NEG, leaky(ai_s + aj1))
        a2 = jnp.where(nbr2 == vself, NEG, leaky(ai_s + aj2))
        a_s = leaky(ai_s + aj_s)

        def allred(v, op):
            for s in (8, 4, 2, 1):
                v = op(v, _dyn_gather(v, jnp.bitwise_xor(iota, s)))
            return v

        m = allred(jnp.maximum(jnp.maximum(a1, a2), a_s), jnp.maximum)
        e1 = jnp.exp(a1 - m)
        e2 = jnp.exp(a2 - m)
        e_s = jnp.exp(a_s - m)
        den = allred(e1 + e2, jnp.add) + e_s + 1e-16
        p1 = e1 / den
        p2 = e2 / den
        p_s = e_s / den
        for g in range(4):
            acc = p_s * gbuf[base_r, pl.ds(g * 16, 16)]
            for k in range(16):
                pk = _dyn_gather(p1, iota * 0 + k)
                acc = acc + pk * gbuf[base_r + 1 + k, pl.ds(g * 16, 16)]
            for k in range(16):
                pk = _dyn_gather(p2, iota * 0 + k)
                acc = acc + pk * gbuf[base_r + 17 + k, pl.ds(g * 16, 16)]
            obuf[n, pl.ds(g * 16, 16)] = acc

    def outer(t, carry):
        for b in range(2):
            c = 2 * t + b
            for cp in gather_copies(c, b):
                cp.wait()
            for n in range(CH):
                compute_node(c, b, n)
            pltpu.sync_copy(obuf, out_hbm.at[pl.ds(base + c * CH, CH)])

            @pl.when(c + 2 < CHN)
            def _():
                for cp in gather_copies(c + 2, b):
                    cp.start()
        return carry

    lax.fori_loop(0, CHN // 2, outer, 0)


def _agg_sparsecore(xa, aj_arr, idx_flat):
    mesh = plsc.VectorSubcoreMesh(core_axis_name="c", subcore_axis_name="s")
    k = functools.partial(
        pl.kernel, mesh=mesh,
        out_type=jax.ShapeDtypeStruct((BNP, 64), jnp.float32),
        scratch_types=[
            pltpu.VMEM((NODES_W * ROWW,), jnp.int32),
            pltpu.VMEM((2 * RPC, XCOL), jnp.float32),
            pltpu.VMEM((2 * RPC,), jnp.float32),
            pltpu.VMEM((CH, 64), jnp.float32),
            pltpu.SemaphoreType.DMA,
            pltpu.SemaphoreType.DMA,
        ],
    )(_agg_sc_body)
    return k(xa, aj_arr, idx_flat)


def kernel(data, emb_W, lin_W, att_i, att_j, att_em_i, att_em_j, bias_gnn,
           bn1_gamma, bn1_beta, bn_out_gamma, bn_out_beta, out_W, out_b):
    topk_idx = _cos_topk(emb_W)

    x = data.reshape(BN, Fw) @ lin_W
    av_i = att_i.reshape(D)
    av_j = att_j.reshape(D)
    ae_i = att_em_i.reshape(D)
    ae_j = att_em_j.reshape(D)
    emb_ai = emb_W @ ae_i
    emb_aj = emb_W @ ae_j
    ai = x @ av_i + jnp.tile(emb_ai, B)
    aj = x @ av_j + jnp.tile(emb_aj, B)

    xa = jnp.concatenate(
        [x, aj[:, None], ai[:, None], jnp.zeros((BN, XCOL - D - 2), jnp.float32)],
        axis=1)
    self_ids = jnp.arange(BN, dtype=jnp.int32)
    nbr = jnp.concatenate([topk_idx + b * N for b in range(B)], axis=0)
    idxm = jnp.concatenate(
        [self_ids[:, None], nbr.astype(jnp.int32),
         jnp.tile(self_ids[:, None], (1, ROWW - 1 - K))], axis=1)
    pad_rows = jnp.tile((jnp.arange(BNP - BN, dtype=jnp.int32) * 37 % BN)[:, None],
                        (1, ROWW))
    idx_flat = jnp.concatenate([idxm, pad_rows], axis=0).reshape(-1)
    agg = _agg_sparsecore(xa, aj, idx_flat)[:BN]

    emb2 = jnp.tile(emb_W, (B, 1))
    o = _postprocess(agg, emb2, bias_gnn, bn1_gamma, bn1_beta,
                     bn_out_gamma, bn_out_beta, out_W, out_b)
    return o.reshape(B, N)


# speedup vs baseline: 2.8383x; 1.0007x over previous
import functools

import jax
import jax.numpy as jnp
from jax import lax
from jax.experimental import pallas as pl
from jax.experimental.pallas import tpu as pltpu
from jax.experimental.pallas import tpu_sc as plsc

B, N, Fw, D, K, H = 2, 10000, 64, 64, 32, 1
BN = B * N
EPS = 1e-5
ROWS_T = 256
NP = 10112
NEG = -3.0e38


def _graph_body(wt_ref, wall_ref, nt_ref, nall_ref, idx_ref, s_ref):
    s = jax.lax.dot_general(wt_ref[...], wall_ref[...],
                            (((1,), (1,)), ((), ())),
                            preferred_element_type=jnp.float32)
    s = s / (nt_ref[...] * nall_ref[...])
    lane = jax.lax.broadcasted_iota(jnp.int32, (ROWS_T, NP), 1)
    s_ref[...] = jnp.where(lane < N, s, NEG)

    def body(k, acc):
        sv = s_ref[...]
        m = jnp.max(sv, axis=1, keepdims=True)
        idx = jnp.min(jnp.where(sv == m, lane, NP), axis=1, keepdims=True)
        acc = jnp.where(lane[:, :128] == k, idx, acc)
        s_ref[...] = jnp.where(lane == idx, NEG, sv)
        return acc

    acc = jax.lax.fori_loop(0, K, body,
                            jnp.zeros((ROWS_T, 128), jnp.int32))
    idx_ref[...] = acc[:, :K]


def _cos_topk(emb_W):
    nrm = jnp.linalg.norm(emb_W, axis=-1)
    wall = jnp.pad(emb_W, ((0, NP - N), (0, 0)))
    nall = jnp.pad(nrm, (0, NP - N), constant_values=1.0).reshape(1, NP)
    ncol = nrm.reshape(N, 1)
    grid = N // ROWS_T + (1 if N % ROWS_T else 0)
    wpad = jnp.pad(emb_W, ((0, grid * ROWS_T - N), (0, 0)))
    npad = jnp.pad(ncol, ((0, grid * ROWS_T - N), (0, 0)), constant_values=1.0)
    idx = pl.pallas_call(
        _graph_body,
        grid=(grid,),
        in_specs=[
            pl.BlockSpec((ROWS_T, D), lambda i: (i, 0)),
            pl.BlockSpec((NP, D), lambda i: (0, 0)),
            pl.BlockSpec((ROWS_T, 1), lambda i: (i, 0)),
            pl.BlockSpec((1, NP), lambda i: (0, 0)),
        ],
        out_specs=pl.BlockSpec((ROWS_T, K), lambda i: (i, 0)),
        out_shape=jax.ShapeDtypeStruct((grid * ROWS_T, K), jnp.int32),
        scratch_shapes=[pltpu.VMEM((ROWS_T, NP), jnp.float32)],
        compiler_params=pltpu.CompilerParams(
            dimension_semantics=("parallel",)),
    )(wpad, wall, npad, nall)
    return idx[:N]


def _post_body(agg_ref, emb2_ref, bias_ref, g1_ref, b1_ref, g2_ref, b2_ref,
               ow_ref, ob_ref, out_ref):
    out = agg_ref[...] + bias_ref[...]
    mu = jnp.mean(out, axis=0, keepdims=True)
    var = jnp.mean((out - mu) ** 2, axis=0, keepdims=True)
    out = (out - mu) * jax.lax.rsqrt(var + EPS) * g1_ref[...] + b1_ref[...]
    out = jnp.maximum(out, 0.0)
    x3 = out * emb2_ref[...]
    mu2 = jnp.mean(x3, axis=0, keepdims=True)
    var2 = jnp.mean((x3 - mu2) ** 2, axis=0, keepdims=True)
    t = (x3 - mu2) * jax.lax.rsqrt(var2 + EPS) * g2_ref[...] + b2_ref[...]
    t = jnp.maximum(t, 0.0)
    o = jnp.sum(t * ow_ref[...], axis=1) + ob_ref[0, 0]
    out_ref[...] = o


def _postprocess(agg, emb2, bias_gnn, g1, b1, g2, b2, out_W, out_b):
    return pl.pallas_call(
        _post_body,
        out_shape=jax.ShapeDtypeStruct((BN,), jnp.float32),
    )(agg, emb2, bias_gnn.reshape(1, D), g1.reshape(1, D), b1.reshape(1, D),
      g2.reshape(1, D), b2.reshape(1, D), out_W.reshape(1, D),
      out_b.reshape(1, 1))


NW = 32
NODES_W = 640
BNP = NW * NODES_W
CH = 8
CHN = NODES_W // CH
ROWW = 40
RPC = CH * ROWW
XCOL = 128


def _dyn_gather(v, idx):
    dnums = lax.GatherDimensionNumbers(
        offset_dims=(), collapsed_slice_dims=(0,), start_index_map=(0,))
    return lax.gather(v, idx[:, None], dnums, slice_sizes=(1,),
                      mode=lax.GatherScatterMode.PROMISE_IN_BOUNDS)


def _agg_sc_body(xa_hbm, aj_hbm, idx_hbm, out_hbm,
                 idx_v, gbuf, ajbuf, obuf, gs0, gs1):
    wid = lax.axis_index("s") * 2 + lax.axis_index("c")
    base = wid * NODES_W
    pltpu.sync_copy(idx_hbm.at[pl.ds(base * ROWW, NODES_W * ROWW)], idx_v)
    gsems = [gs0, gs1]
    iota = lax.iota(jnp.int32, 16)

    def gather_copies(c, b):
        isl = idx_v.at[pl.ds(c * RPC, RPC)]
        return (pltpu.make_async_copy(xa_hbm.at[isl],
                                      gbuf.at[pl.ds(b * RPC, RPC)], gsems[b]),
                pltpu.make_async_copy(aj_hbm.at[isl],
                                      ajbuf.at[pl.ds(b * RPC, RPC)], gsems[b]))

    for b in range(2):
        for cp in gather_copies(b, b):
            cp.start()

    def compute_node(c, b, n):
        base_r = b * RPC + n * ROWW
        ibase = c * RPC + n * ROWW

        vself = _dyn_gather(idx_v[pl.ds(ibase, 16)], iota * 0)
        nbr1 = idx_v[pl.ds(ibase + 1, 16)]
        nbr2 = idx_v[pl.ds(ibase + 17, 16)]
        aj_s = _dyn_gather(ajbuf[pl.ds(base_r, 16)], iota * 0)
        aj1 = ajbuf[pl.ds(base_r + 1, 16)]
        aj2 = ajbuf[pl.ds(base_r + 17, 16)]
        ai_s = _dyn_gather(gbuf[base_r, pl.ds(64, 16)], iota * 0 + 1)

        def leaky(a):
            return jnp.where(a >= 0, a, 0.2 * a)

        a1 = jnp.where(nbr1 == vself, NEG, leaky(ai_s + aj1))
        a2 = jnp.where(nbr2 == vself, NEG, leaky(ai_s + aj2))
        a_s = leaky(ai_s + aj_s)

        def allred(v, op):
            for s in (8, 4, 2, 1):
                v = op(v, _dyn_gather(v, jnp.bitwise_xor(iota, s)))
            return v

        m = allred(jnp.maximum(jnp.maximum(a1, a2), a_s), jnp.maximum)
        e1 = jnp.exp(a1 - m)
        e2 = jnp.exp(a2 - m)
        e_s = jnp.exp(a_s - m)
        den = allred(e1 + e2, jnp.add) + e_s + 1e-16
        p1 = e1 / den
        p2 = e2 / den
        p_s = e_s / den
        for g in range(4):
            acc = p_s * gbuf[base_r, pl.ds(g * 16, 16)]
            for k in range(16):
                pk = _dyn_gather(p1, iota * 0 + k)
                acc = acc + pk * gbuf[base_r + 1 + k, pl.ds(g * 16, 16)]
            for k in range(16):
                pk = _dyn_gather(p2, iota * 0 + k)
                acc = acc + pk * gbuf[base_r + 17 + k, pl.ds(g * 16, 16)]
            obuf[n, pl.ds(g * 16, 16)] = acc

    def outer(t, carry):
        for b in range(2):
            c = 2 * t + b
            for cp in gather_copies(c, b):
                cp.wait()
            for n in range(CH):
                compute_node(c, b, n)
            pltpu.sync_copy(obuf, out_hbm.at[pl.ds(base + c * CH, CH)])

            @pl.when(c + 2 < CHN)
            def _():
                for cp in gather_copies(c + 2, b):
                    cp.start()
        return carry

    lax.fori_loop(0, CHN // 2, outer, 0)


def _agg_sparsecore(xa, aj_arr, idx_flat):
    mesh = plsc.VectorSubcoreMesh(core_axis_name="c", subcore_axis_name="s")
    k = functools.partial(
        pl.kernel, mesh=mesh,
        out_type=jax.ShapeDtypeStruct((BNP, 64), jnp.float32),
        scratch_types=[
            pltpu.VMEM((NODES_W * ROWW,), jnp.int32),
            pltpu.VMEM((2 * RPC, XCOL), jnp.float32),
            pltpu.VMEM((2 * RPC,), jnp.float32),
            pltpu.VMEM((CH, 64), jnp.float32),
            pltpu.SemaphoreType.DMA,
            pltpu.SemaphoreType.DMA,
        ],
    )(_agg_sc_body)
    return k(xa, aj_arr, idx_flat)


def kernel(data, emb_W, lin_W, att_i, att_j, att_em_i, att_em_j, bias_gnn,
           bn1_gamma, bn1_beta, bn_out_gamma, bn_out_beta, out_W, out_b):
    topk_idx = _cos_topk(emb_W)

    x = data.reshape(BN, Fw) @ lin_W
    av_i = att_i.reshape(D)
    av_j = att_j.reshape(D)
    ae_i = att_em_i.reshape(D)
    ae_j = att_em_j.reshape(D)
    emb_ai = emb_W @ ae_i
    emb_aj = emb_W @ ae_j
    ai = x @ av_i + jnp.tile(emb_ai, B)
    aj = x @ av_j + jnp.tile(emb_aj, B)

    xa = jnp.concatenate(
        [x, aj[:, None], ai[:, None], jnp.zeros((BN, XCOL - D - 2), jnp.float32)],
        axis=1)
    self_ids = jnp.arange(BN, dtype=jnp.int32)
    nbr = jnp.concatenate([topk_idx + b * N for b in range(B)], axis=0)
    idxm = jnp.concatenate(
        [self_ids[:, None], nbr.astype(jnp.int32),
         jnp.tile(self_ids[:, None], (1, ROWW - 1 - K))], axis=1)
    pad_rows = jnp.tile((jnp.arange(BNP - BN, dtype=jnp.int32) * 37 % BN)[:, None],
                        (1, ROWW))
    idx_flat = jnp.concatenate([idxm, pad_rows], axis=0).reshape(-1)
    agg = _agg_sparsecore(xa, aj, idx_flat)[:BN]

    emb2 = jnp.tile(emb_W, (B, 1))
    o = _postprocess(agg, emb2, bias_gnn, bn1_gamma, bn1_beta,
                     bn_out_gamma, bn_out_beta, out_W, out_b)
    return o.reshape(B, N)


# speedup vs baseline: 2.9631x; 1.0440x over previous
import functools

import jax
import jax.numpy as jnp
from jax import lax
from jax.experimental import pallas as pl
from jax.experimental.pallas import tpu as pltpu
from jax.experimental.pallas import tpu_sc as plsc

B, N, Fw, D, K, H = 2, 10000, 64, 64, 32, 1
BN = B * N
EPS = 1e-5
ROWS_T = 512
NP = 10112
NEG = -3.0e38


def _graph_body(wt_ref, wall_ref, nt_ref, nall_ref, idx_ref, s_ref):
    s = jax.lax.dot_general(wt_ref[...], wall_ref[...],
                            (((1,), (1,)), ((), ())),
                            preferred_element_type=jnp.float32)
    s = s / (nt_ref[...] * nall_ref[...])
    lane = jax.lax.broadcasted_iota(jnp.int32, (ROWS_T, NP), 1)
    s_ref[...] = jnp.where(lane < N, s, NEG)

    def body(k, acc):
        sv = s_ref[...]
        m = jnp.max(sv, axis=1, keepdims=True)
        idx = jnp.min(jnp.where(sv == m, lane, NP), axis=1, keepdims=True)
        acc = jnp.where(lane[:, :128] == k, idx, acc)
        s_ref[...] = jnp.where(lane == idx, NEG, sv)
        return acc

    acc = jax.lax.fori_loop(0, K, body,
                            jnp.zeros((ROWS_T, 128), jnp.int32))
    idx_ref[...] = acc[:, :K]


def _cos_topk(emb_W):
    nrm = jnp.linalg.norm(emb_W, axis=-1)
    wall = jnp.pad(emb_W, ((0, NP - N), (0, 0)))
    nall = jnp.pad(nrm, (0, NP - N), constant_values=1.0).reshape(1, NP)
    ncol = nrm.reshape(N, 1)
    grid = N // ROWS_T + (1 if N % ROWS_T else 0)
    wpad = jnp.pad(emb_W, ((0, grid * ROWS_T - N), (0, 0)))
    npad = jnp.pad(ncol, ((0, grid * ROWS_T - N), (0, 0)), constant_values=1.0)
    idx = pl.pallas_call(
        _graph_body,
        grid=(grid,),
        in_specs=[
            pl.BlockSpec((ROWS_T, D), lambda i: (i, 0)),
            pl.BlockSpec((NP, D), lambda i: (0, 0)),
            pl.BlockSpec((ROWS_T, 1), lambda i: (i, 0)),
            pl.BlockSpec((1, NP), lambda i: (0, 0)),
        ],
        out_specs=pl.BlockSpec((ROWS_T, K), lambda i: (i, 0)),
        out_shape=jax.ShapeDtypeStruct((grid * ROWS_T, K), jnp.int32),
        scratch_shapes=[pltpu.VMEM((ROWS_T, NP), jnp.float32)],
        compiler_params=pltpu.CompilerParams(
            dimension_semantics=("parallel",)),
    )(wpad, wall, npad, nall)
    return idx[:N]


def _post_body(agg_ref, emb2_ref, bias_ref, g1_ref, b1_ref, g2_ref, b2_ref,
               ow_ref, ob_ref, out_ref):
    out = agg_ref[...] + bias_ref[...]
    mu = jnp.mean(out, axis=0, keepdims=True)
    var = jnp.mean((out - mu) ** 2, axis=0, keepdims=True)
    out = (out - mu) * jax.lax.rsqrt(var + EPS) * g1_ref[...] + b1_ref[...]
    out = jnp.maximum(out, 0.0)
    x3 = out * emb2_ref[...]
    mu2 = jnp.mean(x3, axis=0, keepdims=True)
    var2 = jnp.mean((x3 - mu2) ** 2, axis=0, keepdims=True)
    t = (x3 - mu2) * jax.lax.rsqrt(var2 + EPS) * g2_ref[...] + b2_ref[...]
    t = jnp.maximum(t, 0.0)
    o = jnp.sum(t * ow_ref[...], axis=1) + ob_ref[0, 0]
    out_ref[...] = o


def _postprocess(agg, emb2, bias_gnn, g1, b1, g2, b2, out_W, out_b):
    return pl.pallas_call(
        _post_body,
        out_shape=jax.ShapeDtypeStruct((BN,), jnp.float32),
    )(agg, emb2, bias_gnn.reshape(1, D), g1.reshape(1, D), b1.reshape(1, D),
      g2.reshape(1, D), b2.reshape(1, D), out_W.reshape(1, D),
      out_b.reshape(1, 1))


NW = 32
NODES_W = 640
BNP = NW * NODES_W
CH = 8
CHN = NODES_W // CH
ROWW = 40
RPC = CH * ROWW
XCOL = 128


def _dyn_gather(v, idx):
    dnums = lax.GatherDimensionNumbers(
        offset_dims=(), collapsed_slice_dims=(0,), start_index_map=(0,))
    return lax.gather(v, idx[:, None], dnums, slice_sizes=(1,),
                      mode=lax.GatherScatterMode.PROMISE_IN_BOUNDS)


def _agg_sc_body(xa_hbm, aj_hbm, idx_hbm, out_hbm,
                 idx_v, gbuf, ajbuf, obuf, gs0, gs1):
    wid = lax.axis_index("s") * 2 + lax.axis_index("c")
    base = wid * NODES_W
    pltpu.sync_copy(idx_hbm.at[pl.ds(base * ROWW, NODES_W * ROWW)], idx_v)
    gsems = [gs0, gs1]
    iota = lax.iota(jnp.int32, 16)

    def gather_copies(c, b):
        isl = idx_v.at[pl.ds(c * RPC, RPC)]
        return (pltpu.make_async_copy(xa_hbm.at[isl],
                                      gbuf.at[pl.ds(b * RPC, RPC)], gsems[b]),
                pltpu.make_async_copy(aj_hbm.at[isl],
                                      ajbuf.at[pl.ds(b * RPC, RPC)], gsems[b]))

    for b in range(2):
        for cp in gather_copies(b, b):
            cp.start()

    def compute_node(c, b, n):
        base_r = b * RPC + n * ROWW
        ibase = c * RPC + n * ROWW

        vself = _dyn_gather(idx_v[pl.ds(ibase, 16)], iota * 0)
        nbr1 = idx_v[pl.ds(ibase + 1, 16)]
        nbr2 = idx_v[pl.ds(ibase + 17, 16)]
        aj_s = _dyn_gather(ajbuf[pl.ds(base_r, 16)], iota * 0)
        aj1 = ajbuf[pl.ds(base_r + 1, 16)]
        aj2 = ajbuf[pl.ds(base_r + 17, 16)]
        ai_s = _dyn_gather(gbuf[base_r, pl.ds(64, 16)], iota * 0 + 1)

        def leaky(a):
            return jnp.where(a >= 0, a, 0.2 * a)

        a1 = jnp.where(nbr1 == vself, NEG, leaky(ai_s + aj1))
        a2 = jnp.where(nbr2 == vself, NEG, leaky(ai_s + aj2))
        a_s = leaky(ai_s + aj_s)

        def allred(v, op):
            for s in (8, 4, 2, 1):
                v = op(v, _dyn_gather(v, jnp.bitwise_xor(iota, s)))
            return v

        m = allred(jnp.maximum(jnp.maximum(a1, a2), a_s), jnp.maximum)
        e1 = jnp.exp(a1 - m)
        e2 = jnp.exp(a2 - m)
        e_s = jnp.exp(a_s - m)
        den = allred(e1 + e2, jnp.add) + e_s + 1e-16
        p1 = e1 / den
        p2 = e2 / den
        p_s = e_s / den
        for g in range(4):
            acc = p_s * gbuf[base_r, pl.ds(g * 16, 16)]
            for k in range(16):
                pk = _dyn_gather(p1, iota * 0 + k)
                acc = acc + pk * gbuf[base_r + 1 + k, pl.ds(g * 16, 16)]
            for k in range(16):
                pk = _dyn_gather(p2, iota * 0 + k)
                acc = acc + pk * gbuf[base_r + 17 + k, pl.ds(g * 16, 16)]
            obuf[n, pl.ds(g * 16, 16)] = acc

    def outer(t, carry):
        for b in range(2):
            c = 2 * t + b
            for cp in gather_copies(c, b):
                cp.wait()
            for n in range(CH):
                compute_node(c, b, n)
            pltpu.sync_copy(obuf, out_hbm.at[pl.ds(base + c * CH, CH)])

            @pl.when(c + 2 < CHN)
            def _():
                for cp in gather_copies(c + 2, b):
                    cp.start()
        return carry

    lax.fori_loop(0, CHN // 2, outer, 0)


def _agg_sparsecore(xa, aj_arr, idx_flat):
    mesh = plsc.VectorSubcoreMesh(core_axis_name="c", subcore_axis_name="s")
    k = functools.partial(
        pl.kernel, mesh=mesh,
        out_type=jax.ShapeDtypeStruct((BNP, 64), jnp.float32),
        scratch_types=[
            pltpu.VMEM((NODES_W * ROWW,), jnp.int32),
            pltpu.VMEM((2 * RPC, XCOL), jnp.float32),
            pltpu.VMEM((2 * RPC,), jnp.float32),
            pltpu.VMEM((CH, 64), jnp.float32),
            pltpu.SemaphoreType.DMA,
            pltpu.SemaphoreType.DMA,
        ],
    )(_agg_sc_body)
    return k(xa, aj_arr, idx_flat)


def kernel(data, emb_W, lin_W, att_i, att_j, att_em_i, att_em_j, bias_gnn,
           bn1_gamma, bn1_beta, bn_out_gamma, bn_out_beta, out_W, out_b):
    topk_idx = _cos_topk(emb_W)

    x = data.reshape(BN, Fw) @ lin_W
    av_i = att_i.reshape(D)
    av_j = att_j.reshape(D)
    ae_i = att_em_i.reshape(D)
    ae_j = att_em_j.reshape(D)
    emb_ai = emb_W @ ae_i
    emb_aj = emb_W @ ae_j
    ai = x @ av_i + jnp.tile(emb_ai, B)
    aj = x @ av_j + jnp.tile(emb_aj, B)

    xa = jnp.concatenate(
        [x, aj[:, None], ai[:, None], jnp.zeros((BN, XCOL - D - 2), jnp.float32)],
        axis=1)
    self_ids = jnp.arange(BN, dtype=jnp.int32)
    nbr = jnp.concatenate([topk_idx + b * N for b in range(B)], axis=0)
    idxm = jnp.concatenate(
        [self_ids[:, None], nbr.astype(jnp.int32),
         jnp.tile(self_ids[:, None], (1, ROWW - 1 - K))], axis=1)
    pad_rows = jnp.tile((jnp.arange(BNP - BN, dtype=jnp.int32) * 37 % BN)[:, None],
                        (1, ROWW))
    idx_flat = jnp.concatenate([idxm, pad_rows], axis=0).reshape(-1)
    agg = _agg_sparsecore(xa, aj, idx_flat)[:BN]

    emb2 = jnp.tile(emb_W, (B, 1))
    o = _postprocess(agg, emb2, bias_gnn, bn1_gamma, bn1_beta,
                     bn_out_gamma, bn_out_beta, out_W, out_b)
    return o.reshape(B, N)


# speedup vs baseline: 3.0249x; 1.0209x over previous
import functools

import jax
import jax.numpy as jnp
from jax import lax
from jax.experimental import pallas as pl
from jax.experimental.pallas import tpu as pltpu
from jax.experimental.pallas import tpu_sc as plsc

B, N, Fw, D, K, H = 2, 10000, 64, 64, 32, 1
BN = B * N
EPS = 1e-5
ROWS_T = 1024
NP = 10112
NEG = -3.0e38


def _graph_body(wt_ref, wall_ref, nt_ref, nall_ref, idx_ref, s_ref):
    s = jax.lax.dot_general(wt_ref[...], wall_ref[...],
                            (((1,), (1,)), ((), ())),
                            preferred_element_type=jnp.float32)
    s = s / (nt_ref[...] * nall_ref[...])
    lane = jax.lax.broadcasted_iota(jnp.int32, (ROWS_T, NP), 1)
    s_ref[...] = jnp.where(lane < N, s, NEG)

    def body(k, acc):
        sv = s_ref[...]
        m = jnp.max(sv, axis=1, keepdims=True)
        idx = jnp.min(jnp.where(sv == m, lane, NP), axis=1, keepdims=True)
        acc = jnp.where(lane[:, :128] == k, idx, acc)
        s_ref[...] = jnp.where(lane == idx, NEG, sv)
        return acc

    acc = jax.lax.fori_loop(0, K, body,
                            jnp.zeros((ROWS_T, 128), jnp.int32))
    idx_ref[...] = acc[:, :K]


def _cos_topk(emb_W):
    nrm = jnp.linalg.norm(emb_W, axis=-1)
    wall = jnp.pad(emb_W, ((0, NP - N), (0, 0)))
    nall = jnp.pad(nrm, (0, NP - N), constant_values=1.0).reshape(1, NP)
    ncol = nrm.reshape(N, 1)
    grid = N // ROWS_T + (1 if N % ROWS_T else 0)
    wpad = jnp.pad(emb_W, ((0, grid * ROWS_T - N), (0, 0)))
    npad = jnp.pad(ncol, ((0, grid * ROWS_T - N), (0, 0)), constant_values=1.0)
    idx = pl.pallas_call(
        _graph_body,
        grid=(grid,),
        in_specs=[
            pl.BlockSpec((ROWS_T, D), lambda i: (i, 0)),
            pl.BlockSpec((NP, D), lambda i: (0, 0)),
            pl.BlockSpec((ROWS_T, 1), lambda i: (i, 0)),
            pl.BlockSpec((1, NP), lambda i: (0, 0)),
        ],
        out_specs=pl.BlockSpec((ROWS_T, K), lambda i: (i, 0)),
        out_shape=jax.ShapeDtypeStruct((grid * ROWS_T, K), jnp.int32),
        scratch_shapes=[pltpu.VMEM((ROWS_T, NP), jnp.float32)],
        compiler_params=pltpu.CompilerParams(
            dimension_semantics=("parallel",)),
    )(wpad, wall, npad, nall)
    return idx[:N]


def _post_body(agg_ref, emb2_ref, bias_ref, g1_ref, b1_ref, g2_ref, b2_ref,
               ow_ref, ob_ref, out_ref):
    out = agg_ref[...] + bias_ref[...]
    mu = jnp.mean(out, axis=0, keepdims=True)
    var = jnp.mean((out - mu) ** 2, axis=0, keepdims=True)
    out = (out - mu) * jax.lax.rsqrt(var + EPS) * g1_ref[...] + b1_ref[...]
    out = jnp.maximum(out, 0.0)
    x3 = out * emb2_ref[...]
    mu2 = jnp.mean(x3, axis=0, keepdims=True)
    var2 = jnp.mean((x3 - mu2) ** 2, axis=0, keepdims=True)
    t = (x3 - mu2) * jax.lax.rsqrt(var2 + EPS) * g2_ref[...] + b2_ref[...]
    t = jnp.maximum(t, 0.0)
    o = jnp.sum(t * ow_ref[...], axis=1) + ob_ref[0, 0]
    out_ref[...] = o


def _postprocess(agg, emb2, bias_gnn, g1, b1, g2, b2, out_W, out_b):
    return pl.pallas_call(
        _post_body,
        out_shape=jax.ShapeDtypeStruct((BN,), jnp.float32),
    )(agg, emb2, bias_gnn.reshape(1, D), g1.reshape(1, D), b1.reshape(1, D),
      g2.reshape(1, D), b2.reshape(1, D), out_W.reshape(1, D),
      out_b.reshape(1, 1))


NW = 32
NODES_W = 640
BNP = NW * NODES_W
CH = 8
CHN = NODES_W // CH
ROWW = 40
RPC = CH * ROWW
XCOL = 128


def _dyn_gather(v, idx):
    dnums = lax.GatherDimensionNumbers(
        offset_dims=(), collapsed_slice_dims=(0,), start_index_map=(0,))
    return lax.gather(v, idx[:, None], dnums, slice_sizes=(1,),
                      mode=lax.GatherScatterMode.PROMISE_IN_BOUNDS)


def _agg_sc_body(xa_hbm, aj_hbm, idx_hbm, out_hbm,
                 idx_v, gbuf, ajbuf, obuf, gs0, gs1):
    wid = lax.axis_index("s") * 2 + lax.axis_index("c")
    base = wid * NODES_W
    pltpu.sync_copy(idx_hbm.at[pl.ds(base * ROWW, NODES_W * ROWW)], idx_v)
    gsems = [gs0, gs1]
    iota = lax.iota(jnp.int32, 16)

    def gather_copies(c, b):
        isl = idx_v.at[pl.ds(c * RPC, RPC)]
        return (pltpu.make_async_copy(xa_hbm.at[isl],
                                      gbuf.at[pl.ds(b * RPC, RPC)], gsems[b]),
                pltpu.make_async_copy(aj_hbm.at[isl],
                                      ajbuf.at[pl.ds(b * RPC, RPC)], gsems[b]))

    for b in range(2):
        for cp in gather_copies(b, b):
            cp.start()

    def compute_node(c, b, n):
        base_r = b * RPC + n * ROWW
        ibase = c * RPC + n * ROWW

        vself = _dyn_gather(idx_v[pl.ds(ibase, 16)], iota * 0)
        nbr1 = idx_v[pl.ds(ibase + 1, 16)]
        nbr2 = idx_v[pl.ds(ibase + 17, 16)]
        aj_s = _dyn_gather(ajbuf[pl.ds(base_r, 16)], iota * 0)
        aj1 = ajbuf[pl.ds(base_r + 1, 16)]
        aj2 = ajbuf[pl.ds(base_r + 17, 16)]
        ai_s = _dyn_gather(gbuf[base_r, pl.ds(64, 16)], iota * 0 + 1)

        def leaky(a):
            return jnp.where(a >= 0, a, 0.2 * a)

        a1 = jnp.where(nbr1 == vself, NEG, leaky(ai_s + aj1))
        a2 = jnp.where(nbr2 == vself, NEG, leaky(ai_s + aj2))
        a_s = leaky(ai_s + aj_s)

        def allred(v, op):
            for s in (8, 4, 2, 1):
                v = op(v, _dyn_gather(v, jnp.bitwise_xor(iota, s)))
            return v

        m = allred(jnp.maximum(jnp.maximum(a1, a2), a_s), jnp.maximum)
        e1 = jnp.exp(a1 - m)
        e2 = jnp.exp(a2 - m)
        e_s = jnp.exp(a_s - m)
        den = allred(e1 + e2, jnp.add) + e_s + 1e-16
        p1 = e1 / den
        p2 = e2 / den
        p_s = e_s / den
        for g in range(4):
            acc = p_s * gbuf[base_r, pl.ds(g * 16, 16)]
            for k in range(16):
                pk = _dyn_gather(p1, iota * 0 + k)
                acc = acc + pk * gbuf[base_r + 1 + k, pl.ds(g * 16, 16)]
            for k in range(16):
                pk = _dyn_gather(p2, iota * 0 + k)
                acc = acc + pk * gbuf[base_r + 17 + k, pl.ds(g * 16, 16)]
            obuf[n, pl.ds(g * 16, 16)] = acc

    def outer(t, carry):
        for b in range(2):
            c = 2 * t + b
            for cp in gather_copies(c, b):
                cp.wait()
            for n in range(CH):
                compute_node(c, b, n)
            pltpu.sync_copy(obuf, out_hbm.at[pl.ds(base + c * CH, CH)])

            @pl.when(c + 2 < CHN)
            def _():
                for cp in gather_copies(c + 2, b):
                    cp.start()
        return carry

    lax.fori_loop(0, CHN // 2, outer, 0)


def _agg_sparsecore(xa, aj_arr, idx_flat):
    mesh = plsc.VectorSubcoreMesh(core_axis_name="c", subcore_axis_name="s")
    k = functools.partial(
        pl.kernel, mesh=mesh,
        out_type=jax.ShapeDtypeStruct((BNP, 64), jnp.float32),
        scratch_types=[
            pltpu.VMEM((NODES_W * ROWW,), jnp.int32),
            pltpu.VMEM((2 * RPC, XCOL), jnp.float32),
            pltpu.VMEM((2 * RPC,), jnp.float32),
            pltpu.VMEM((CH, 64), jnp.float32),
            pltpu.SemaphoreType.DMA,
            pltpu.SemaphoreType.DMA,
        ],
    )(_agg_sc_body)
    return k(xa, aj_arr, idx_flat)


def kernel(data, emb_W, lin_W, att_i, att_j, att_em_i, att_em_j, bias_gnn,
           bn1_gamma, bn1_beta, bn_out_gamma, bn_out_beta, out_W, out_b):
    topk_idx = _cos_topk(emb_W)

    x = data.reshape(BN, Fw) @ lin_W
    av_i = att_i.reshape(D)
    av_j = att_j.reshape(D)
    ae_i = att_em_i.reshape(D)
    ae_j = att_em_j.reshape(D)
    emb_ai = emb_W @ ae_i
    emb_aj = emb_W @ ae_j
    ai = x @ av_i + jnp.tile(emb_ai, B)
    aj = x @ av_j + jnp.tile(emb_aj, B)

    xa = jnp.concatenate(
        [x, aj[:, None], ai[:, None], jnp.zeros((BN, XCOL - D - 2), jnp.float32)],
        axis=1)
    self_ids = jnp.arange(BN, dtype=jnp.int32)
    nbr = jnp.concatenate([topk_idx + b * N for b in range(B)], axis=0)
    idxm = jnp.concatenate(
        [self_ids[:, None], nbr.astype(jnp.int32),
         jnp.tile(self_ids[:, None], (1, ROWW - 1 - K))], axis=1)
    pad_rows = jnp.tile((jnp.arange(BNP - BN, dtype=jnp.int32) * 37 % BN)[:, None],
                        (1, ROWW))
    idx_flat = jnp.concatenate([idxm, pad_rows], axis=0).reshape(-1)
    agg = _agg_sparsecore(xa, aj, idx_flat)[:BN]

    emb2 = jnp.tile(emb_W, (B, 1))
    o = _postprocess(agg, emb2, bias_gnn, bn1_gamma, bn1_beta,
                     bn_out_gamma, bn_out_beta, out_W, out_b)
    return o.reshape(B, N)
